# Initial kernel scaffold; baseline (speedup 1.0000x reference)
import jax
import jax.numpy as jnp
from jax.experimental import pallas as pl


def kernel(node_feats, edge_index, W_self, b_self, W_neigh, b_neigh):
    raise NotImplementedError("write your pallas kernel here")


# speedup vs baseline: 3.2544x; 3.2544x over previous
import functools

import jax
import jax.numpy as jnp
from jax import lax
from jax.experimental import pallas as pl
from jax.experimental.pallas import tpu as pltpu
from jax.experimental.pallas import tpu_sc as plsc

N = 10000
D = 256
DH = D // 2
E = 160000
NS = 16
EPS = E // NS
K = 80
NCHUNK = EPS // K
NRC = N // K
RCS = NRC // NS + 1
EPC = E // 2
EPS2 = EPC // NS
K2 = 40
NCHUNK2 = EPS2 // K2

_f32 = jnp.float32


@functools.partial(
    pl.kernel,
    out_type=(
        jax.ShapeDtypeStruct((N, DH), _f32),
        jax.ShapeDtypeStruct((N, DH), _f32),
    ),
    mesh=plsc.VectorSubcoreMesh(core_axis_name="c", subcore_axis_name="s"),
    scratch_types=[
        pltpu.VMEM_SHARED((N, DH), _f32),
        pltpu.VMEM((K,), jnp.int32),
        pltpu.VMEM((K,), jnp.int32),
        pltpu.VMEM((K, DH), _f32),
        pltpu.SemaphoreType.DMA,
    ],
)
def _sc_feats(t0_hbm, t1_hbm, src_hbm, dst_hbm, zf_hbm,
              out0_hbm, out1_hbm,
              acc_s, src_v, dst_v, rows_v, sem):
    cid = lax.axis_index("c")
    sid = lax.axis_index("s")

    pltpu.sync_copy(zf_hbm, rows_v)
    for j in range(RCS):
        c = sid + j * NS

        @pl.when(c < NRC)
        def _():
            pltpu.sync_copy(rows_v, acc_s.at[pl.ds(c * K, K)])

    plsc.subcore_barrier()

    ebase = sid * EPS

    def _chunk(i, table_hbm):
        base = ebase + i * K
        pltpu.sync_copy(src_hbm.at[pl.ds(base, K)], src_v)
        pltpu.sync_copy(dst_hbm.at[pl.ds(base, K)], dst_v)
        pltpu.async_copy(table_hbm.at[src_v], rows_v, sem).wait()
        pltpu.sync_copy(rows_v, acc_s.at[dst_v], add=True)

    @pl.when(cid == 0)
    def _():
        for i in range(NCHUNK):
            _chunk(i, t0_hbm)

    @pl.when(cid == 1)
    def _():
        for i in range(NCHUNK):
            _chunk(i, t1_hbm)

    plsc.subcore_barrier()

    def _wb(out_hbm):
        for j in range(RCS):
            c = sid + j * NS

            @pl.when(c < NRC)
            def _():
                pltpu.sync_copy(acc_s.at[pl.ds(c * K, K)], rows_v)
                pltpu.sync_copy(rows_v, out_hbm.at[pl.ds(c * K, K)])

    @pl.when(cid == 0)
    def _():
        _wb(out0_hbm)

    @pl.when(cid == 1)
    def _():
        _wb(out1_hbm)


@functools.partial(
    pl.kernel,
    out_type=(
        jax.ShapeDtypeStruct((N, DH), _f32),
        jax.ShapeDtypeStruct((N, DH), _f32),
    ),
    mesh=plsc.VectorSubcoreMesh(core_axis_name="c", subcore_axis_name="s"),
    scratch_types=[
        pltpu.VMEM_SHARED((N, DH), _f32),
        pltpu.VMEM((K2,), jnp.int32),
        pltpu.VMEM((K2, DH), _f32),
        pltpu.VMEM((K, DH), _f32),
    ],
)
def _sc_degree(dst_hbm, zf_hbm, ones_hbm,
               deg0_hbm, deg1_hbm,
               deg_s, dst_v, ones_v, stage_v):
    cid = lax.axis_index("c")
    sid = lax.axis_index("s")

    pltpu.sync_copy(zf_hbm, stage_v)
    pltpu.sync_copy(ones_hbm, ones_v)
    for j in range(RCS):
        c = sid + j * NS

        @pl.when(c < NRC)
        def _():
            pltpu.sync_copy(stage_v, deg_s.at[pl.ds(c * K, K)])

    plsc.subcore_barrier()

    ebase = cid * EPC + sid * EPS2
    for i in range(NCHUNK2):
        base = ebase + i * K2
        pltpu.sync_copy(dst_hbm.at[pl.ds(base, K2)], dst_v)
        pltpu.sync_copy(ones_v, deg_s.at[dst_v], add=True)

    plsc.subcore_barrier()

    def _wb(out_hbm):
        for j in range(RCS):
            c = sid + j * NS

            @pl.when(c < NRC)
            def _():
                pltpu.sync_copy(deg_s.at[pl.ds(c * K, K)], stage_v)
                pltpu.sync_copy(stage_v, out_hbm.at[pl.ds(c * K, K)])

    @pl.when(cid == 0)
    def _():
        _wb(deg0_hbm)

    @pl.when(cid == 1)
    def _():
        _wb(deg1_hbm)


BLK = 1000


def _tc_body(x_ref, a0_ref, a1_ref, d0_ref, d1_ref, wst_ref, wnt0_ref,
             wnt1_ref, b_ref, o_ref):
    deg = d0_ref[:, 0:1] + d1_ref[:, 0:1]
    inv = 1.0 / jnp.maximum(deg, 1.0)
    m0 = a0_ref[...] * inv
    m1 = a1_ref[...] * inv
    acc = jnp.dot(x_ref[...], wst_ref[...], preferred_element_type=_f32)
    acc = acc + jnp.dot(m0, wnt0_ref[...], preferred_element_type=_f32)
    acc = acc + jnp.dot(m1, wnt1_ref[...], preferred_element_type=_f32)
    o_ref[...] = jnp.maximum(acc + b_ref[...], 0.0)


def _tc_dense(x, a0, a1, d0, d1, wst, wnt0, wnt1, b):
    return pl.pallas_call(
        _tc_body,
        grid=(N // BLK,),
        in_specs=[
            pl.BlockSpec((BLK, D), lambda i: (i, 0)),
            pl.BlockSpec((BLK, DH), lambda i: (i, 0)),
            pl.BlockSpec((BLK, DH), lambda i: (i, 0)),
            pl.BlockSpec((BLK, DH), lambda i: (i, 0)),
            pl.BlockSpec((BLK, DH), lambda i: (i, 0)),
            pl.BlockSpec((D, D), lambda i: (0, 0)),
            pl.BlockSpec((DH, D), lambda i: (0, 0)),
            pl.BlockSpec((DH, D), lambda i: (0, 0)),
            pl.BlockSpec((1, D), lambda i: (0, 0)),
        ],
        out_specs=pl.BlockSpec((BLK, D), lambda i: (i, 0)),
        out_shape=jax.ShapeDtypeStruct((N, D), _f32),
    )(x, a0, a1, d0, d1, wst, wnt0, wnt1, b)


def kernel(node_feats, edge_index, W_self, b_self, W_neigh, b_neigh):
    src = edge_index[0]
    dst = edge_index[1]
    t0 = node_feats[:, :DH]
    t1 = node_feats[:, DH:]
    zf = jnp.zeros((K, DH), _f32)
    ones = jnp.ones((K2, DH), _f32)
    a0, a1 = _sc_feats(t0, t1, src, dst, zf)
    d0, d1 = _sc_degree(dst, zf, ones)
    wst = W_self.T
    wnt = W_neigh.T
    b = (b_self + b_neigh)[None, :]
    return _tc_dense(node_feats, a0, a1, d0, d1, wst, wnt[:DH], wnt[DH:], b)


# speedup vs baseline: 4.3314x; 1.3309x over previous
import functools

import jax
import jax.numpy as jnp
from jax import lax
from jax.experimental import pallas as pl
from jax.experimental.pallas import tpu as pltpu
from jax.experimental.pallas import tpu_sc as plsc

N = 10000
ND = N + 8
D = 256
DH = D // 2
E = 160000
NS = 16
EPS = E // NS
KC = 128
NCH = 80
EPSP = NCH * KC
NZC = N // KC
NZT = N - NZC * KC
EPC2 = E // 2
EPS2 = EPC2 // NS
NCH2 = (EPS2 + KC - 1) // KC
EPS2P = NCH2 * KC
K = 80
NRC = N // K
RCS = NRC // NS + 1

_f32 = jnp.float32


@functools.partial(
    pl.kernel,
    out_type=(
        jax.ShapeDtypeStruct((N, DH), _f32),
        jax.ShapeDtypeStruct((N, DH), _f32),
    ),
    mesh=plsc.VectorSubcoreMesh(core_axis_name="c", subcore_axis_name="s"),
    scratch_types=[
        pltpu.VMEM_SHARED((ND, DH), _f32),
        pltpu.VMEM((3, KC), jnp.int32),
        pltpu.VMEM((3, KC), jnp.int32),
        pltpu.VMEM((KC, DH), _f32),
        pltpu.VMEM((KC, DH), _f32),
        pltpu.SemaphoreType.DMA,
        pltpu.SemaphoreType.DMA,
        pltpu.SemaphoreType.DMA,
        pltpu.SemaphoreType.DMA,
        pltpu.SemaphoreType.DMA,
        pltpu.SemaphoreType.DMA,
        pltpu.SemaphoreType.DMA,
        pltpu.SemaphoreType.DMA,
    ],
)
def _sc_feats(t0_hbm, t1_hbm, src_hbm, dst_hbm, zf_hbm,
              out0_hbm, out1_hbm,
              acc_s, src_c, dst_c, rows_a, rows_b,
              sga, sgb, ss0, ss1, ss2, sd0, sd1, sd2):
    cid = lax.axis_index("c")
    sid = lax.axis_index("s")

    pltpu.sync_copy(zf_hbm, rows_a)
    for j in range(NZC // NS + 1):
        c = sid + j * NS

        @pl.when(c < NZC)
        def _():
            pltpu.sync_copy(rows_a, acc_s.at[pl.ds(c * KC, KC)])

    @pl.when(sid == 0)
    def _():
        pltpu.sync_copy(rows_a.at[pl.ds(0, NZT)], acc_s.at[pl.ds(NZC * KC, NZT)])

    plsc.subcore_barrier()

    gsem = (sga, sgb)
    ssem = (ss0, ss1, ss2)
    dsem = (sd0, sd1, sd2)
    bufs = (rows_a, rows_b)

    def _run(table_hbm):
        hs = [None] * NCH
        hd = [None] * NCH
        hg = [None] * NCH
        pltpu.sync_copy(src_hbm.at[sid, 0], src_c.at[0])
        pltpu.sync_copy(dst_hbm.at[sid, 0], dst_c.at[0])
        hs[1] = pltpu.async_copy(src_hbm.at[sid, 1], src_c.at[1], ssem[1])
        hd[1] = pltpu.async_copy(dst_hbm.at[sid, 1], dst_c.at[1], dsem[1])
        hg[0] = pltpu.async_copy(table_hbm.at[src_c.at[0]], bufs[0], gsem[0])
        for i in range(NCH):
            if i + 1 < NCH:
                hs[i + 1].wait()
                hd[i + 1].wait()
                hg[i + 1] = pltpu.async_copy(
                    table_hbm.at[src_c.at[(i + 1) % 3]], bufs[(i + 1) % 2],
                    gsem[(i + 1) % 2])
            if i + 2 < NCH:
                hs[i + 2] = pltpu.async_copy(
                    src_hbm.at[sid, i + 2], src_c.at[(i + 2) % 3],
                    ssem[(i + 2) % 3])
                hd[i + 2] = pltpu.async_copy(
                    dst_hbm.at[sid, i + 2], dst_c.at[(i + 2) % 3],
                    dsem[(i + 2) % 3])
            hg[i].wait()
            pltpu.sync_copy(bufs[i % 2], acc_s.at[dst_c.at[i % 3]], add=True)

    @pl.when(cid == 0)
    def _():
        _run(t0_hbm)

    @pl.when(cid == 1)
    def _():
        _run(t1_hbm)

    plsc.subcore_barrier()

    def _wb(out_hbm):
        for j in range(NZC // NS + 1):
            c = sid + j * NS

            @pl.when(c < NZC)
            def _():
                pltpu.sync_copy(acc_s.at[pl.ds(c * KC, KC)], rows_a)
                pltpu.sync_copy(rows_a, out_hbm.at[pl.ds(c * KC, KC)])

        @pl.when(sid == 0)
        def _():
            pltpu.sync_copy(acc_s.at[pl.ds(NZC * KC, NZT)], rows_a.at[pl.ds(0, NZT)])
            pltpu.sync_copy(rows_a.at[pl.ds(0, NZT)], out_hbm.at[pl.ds(NZC * KC, NZT)])

    @pl.when(cid == 0)
    def _():
        _wb(out0_hbm)

    @pl.when(cid == 1)
    def _():
        _wb(out1_hbm)


@functools.partial(
    pl.kernel,
    out_type=(
        jax.ShapeDtypeStruct((N, DH), _f32),
        jax.ShapeDtypeStruct((N, DH), _f32),
    ),
    mesh=plsc.VectorSubcoreMesh(core_axis_name="c", subcore_axis_name="s"),
    scratch_types=[
        pltpu.VMEM_SHARED((ND, DH), _f32),
        pltpu.VMEM((NCH2, KC), jnp.int32),
        pltpu.VMEM((KC, DH), _f32),
        pltpu.VMEM((K, DH), _f32),
    ],
)
def _sc_degree(dst_hbm, zf_hbm, ones_hbm,
               deg0_hbm, deg1_hbm,
               deg_s, dst_v, ones_v, stage_v):
    cid = lax.axis_index("c")
    sid = lax.axis_index("s")

    pltpu.sync_copy(dst_hbm.at[cid * NS + sid], dst_v)
    pltpu.sync_copy(zf_hbm, stage_v)
    pltpu.sync_copy(ones_hbm, ones_v)
    for j in range(RCS):
        c = sid + j * NS

        @pl.when(c < NRC)
        def _():
            pltpu.sync_copy(stage_v, deg_s.at[pl.ds(c * K, K)])

    plsc.subcore_barrier()

    for i in range(NCH2):
        pltpu.sync_copy(ones_v, deg_s.at[dst_v.at[i]], add=True)

    plsc.subcore_barrier()

    def _wb(out_hbm):
        for j in range(RCS):
            c = sid + j * NS

            @pl.when(c < NRC)
            def _():
                pltpu.sync_copy(deg_s.at[pl.ds(c * K, K)], stage_v)
                pltpu.sync_copy(stage_v, out_hbm.at[pl.ds(c * K, K)])

    @pl.when(cid == 0)
    def _():
        _wb(deg0_hbm)

    @pl.when(cid == 1)
    def _():
        _wb(deg1_hbm)


BLK = 1000


def _tc_body(x_ref, a0_ref, a1_ref, d0_ref, d1_ref, wst_ref, wnt0_ref,
             wnt1_ref, b_ref, o_ref):
    deg = d0_ref[:, 0:1] + d1_ref[:, 0:1]
    inv = 1.0 / jnp.maximum(deg, 1.0)
    m0 = a0_ref[...] * inv
    m1 = a1_ref[...] * inv
    acc = jnp.dot(x_ref[...], wst_ref[...], preferred_element_type=_f32)
    acc = acc + jnp.dot(m0, wnt0_ref[...], preferred_element_type=_f32)
    acc = acc + jnp.dot(m1, wnt1_ref[...], preferred_element_type=_f32)
    o_ref[...] = jnp.maximum(acc + b_ref[...], 0.0)


def _tc_dense(x, a0, a1, d0, d1, wst, wnt0, wnt1, b):
    return pl.pallas_call(
        _tc_body,
        grid=(N // BLK,),
        in_specs=[
            pl.BlockSpec((BLK, D), lambda i: (i, 0)),
            pl.BlockSpec((BLK, DH), lambda i: (i, 0)),
            pl.BlockSpec((BLK, DH), lambda i: (i, 0)),
            pl.BlockSpec((BLK, DH), lambda i: (i, 0)),
            pl.BlockSpec((BLK, DH), lambda i: (i, 0)),
            pl.BlockSpec((D, D), lambda i: (0, 0)),
            pl.BlockSpec((DH, D), lambda i: (0, 0)),
            pl.BlockSpec((DH, D), lambda i: (0, 0)),
            pl.BlockSpec((1, D), lambda i: (0, 0)),
        ],
        out_specs=pl.BlockSpec((BLK, D), lambda i: (i, 0)),
        out_shape=jax.ShapeDtypeStruct((N, D), _f32),
    )(x, a0, a1, d0, d1, wst, wnt0, wnt1, b)


def kernel(node_feats, edge_index, W_self, b_self, W_neigh, b_neigh):
    src = edge_index[0]
    dst = edge_index[1]
    t0 = node_feats[:, :DH]
    t1 = node_feats[:, DH:]
    src_p = jnp.pad(src.reshape(NS, EPS), ((0, 0), (0, EPSP - EPS))
                    ).reshape(NS, NCH, KC)
    dst_p = jnp.pad(dst.reshape(NS, EPS), ((0, 0), (0, EPSP - EPS)),
                    constant_values=N).reshape(NS, NCH, KC)
    dst_d = jnp.pad(dst.reshape(2 * NS, EPS2), ((0, 0), (0, EPS2P - EPS2)),
                    constant_values=N).reshape(2 * NS, NCH2, KC)
    zf = jnp.zeros((KC, DH), _f32)
    zf80 = jnp.zeros((K, DH), _f32)
    ones = jnp.ones((KC, DH), _f32)
    a0, a1 = _sc_feats(t0, t1, src_p, dst_p, zf)
    d0, d1 = _sc_degree(dst_d, zf80, ones)
    wst = W_self.T
    wnt = W_neigh.T
    b = (b_self + b_neigh)[None, :]
    return _tc_dense(node_feats, a0, a1, d0, d1, wst, wnt[:DH], wnt[DH:], b)


# speedup vs baseline: 4.3331x; 1.0004x over previous
import functools

import jax
import jax.numpy as jnp
from jax import lax
from jax.experimental import pallas as pl
from jax.experimental.pallas import tpu as pltpu
from jax.experimental.pallas import tpu_sc as plsc

N = 10000
ND = N + 8
D = 256
DH = D // 2
E = 160000
NS = 16
EPS = E // NS
KC = 128
NCH = 80
EPSP = NCH * KC
NZC = N // KC
NZT = N - NZC * KC
EPC2 = E // 2
EPS2 = EPC2 // NS
NCH2 = (EPS2 + KC - 1) // KC
EPS2P = NCH2 * KC
K = 80
NRC = N // K
RCS = NRC // NS + 1

_f32 = jnp.float32


@functools.partial(
    pl.kernel,
    out_type=(
        jax.ShapeDtypeStruct((N, DH), _f32),
        jax.ShapeDtypeStruct((N, DH), _f32),
    ),
    mesh=plsc.VectorSubcoreMesh(core_axis_name="c", subcore_axis_name="s"),
    scratch_types=[
        pltpu.VMEM_SHARED((ND, DH), _f32),
        pltpu.VMEM((3, KC), jnp.int32),
        pltpu.VMEM((3, KC), jnp.int32),
        pltpu.VMEM((KC, DH), _f32),
        pltpu.VMEM((KC, DH), _f32),
        pltpu.SemaphoreType.DMA,
        pltpu.SemaphoreType.DMA,
        pltpu.SemaphoreType.DMA,
        pltpu.SemaphoreType.DMA,
        pltpu.SemaphoreType.DMA,
        pltpu.SemaphoreType.DMA,
        pltpu.SemaphoreType.DMA,
        pltpu.SemaphoreType.DMA,
        pltpu.SemaphoreType.DMA,
        pltpu.SemaphoreType.DMA,
    ],
)
def _sc_feats(t0_hbm, t1_hbm, src_hbm, dst_hbm, zf_hbm,
              out0_hbm, out1_hbm,
              acc_s, src_c, dst_c, rows_a, rows_b,
              sga, sgb, ss0, ss1, ss2, sd0, sd1, sd2, sca, scb):
    cid = lax.axis_index("c")
    sid = lax.axis_index("s")

    pltpu.sync_copy(zf_hbm, rows_a)
    for j in range(NZC // NS + 1):
        c = sid + j * NS

        @pl.when(c < NZC)
        def _():
            pltpu.sync_copy(rows_a, acc_s.at[pl.ds(c * KC, KC)])

    @pl.when(sid == 0)
    def _():
        pltpu.sync_copy(rows_a.at[pl.ds(0, NZT)], acc_s.at[pl.ds(NZC * KC, NZT)])

    plsc.subcore_barrier()

    gsem = (sga, sgb)
    ssem = (ss0, ss1, ss2)
    dsem = (sd0, sd1, sd2)
    csem = (sca, scb)
    bufs = (rows_a, rows_b)

    def _run(table_hbm):
        hs = [None] * NCH
        hd = [None] * NCH
        hg = [None] * NCH
        hc = [None] * NCH
        pltpu.sync_copy(src_hbm.at[sid, 0], src_c.at[0])
        pltpu.sync_copy(dst_hbm.at[sid, 0], dst_c.at[0])
        hs[1] = pltpu.async_copy(src_hbm.at[sid, 1], src_c.at[1], ssem[1])
        hd[1] = pltpu.async_copy(dst_hbm.at[sid, 1], dst_c.at[1], dsem[1])
        hg[0] = pltpu.async_copy(table_hbm.at[src_c.at[0]], bufs[0], gsem[0])
        for i in range(NCH):
            if i >= 1:
                hc[i - 1].wait()
            if i + 1 < NCH:
                hs[i + 1].wait()
                hd[i + 1].wait()
                hg[i + 1] = pltpu.async_copy(
                    table_hbm.at[src_c.at[(i + 1) % 3]], bufs[(i + 1) % 2],
                    gsem[(i + 1) % 2])
            if i + 2 < NCH:
                hs[i + 2] = pltpu.async_copy(
                    src_hbm.at[sid, i + 2], src_c.at[(i + 2) % 3],
                    ssem[(i + 2) % 3])
                hd[i + 2] = pltpu.async_copy(
                    dst_hbm.at[sid, i + 2], dst_c.at[(i + 2) % 3],
                    dsem[(i + 2) % 3])
            hg[i].wait()
            hc[i] = pltpu.async_copy(bufs[i % 2], acc_s.at[dst_c.at[i % 3]],
                                     csem[i % 2], add=True)
        hc[NCH - 1].wait()

    @pl.when(cid == 0)
    def _():
        _run(t0_hbm)

    @pl.when(cid == 1)
    def _():
        _run(t1_hbm)

    plsc.subcore_barrier()

    def _wb(out_hbm):
        for j in range(NZC // NS + 1):
            c = sid + j * NS

            @pl.when(c < NZC)
            def _():
                pltpu.sync_copy(acc_s.at[pl.ds(c * KC, KC)], rows_a)
                pltpu.sync_copy(rows_a, out_hbm.at[pl.ds(c * KC, KC)])

        @pl.when(sid == 0)
        def _():
            pltpu.sync_copy(acc_s.at[pl.ds(NZC * KC, NZT)], rows_a.at[pl.ds(0, NZT)])
            pltpu.sync_copy(rows_a.at[pl.ds(0, NZT)], out_hbm.at[pl.ds(NZC * KC, NZT)])

    @pl.when(cid == 0)
    def _():
        _wb(out0_hbm)

    @pl.when(cid == 1)
    def _():
        _wb(out1_hbm)


@functools.partial(
    pl.kernel,
    out_type=(
        jax.ShapeDtypeStruct((N, DH), _f32),
        jax.ShapeDtypeStruct((N, DH), _f32),
    ),
    mesh=plsc.VectorSubcoreMesh(core_axis_name="c", subcore_axis_name="s"),
    scratch_types=[
        pltpu.VMEM_SHARED((ND, DH), _f32),
        pltpu.VMEM((NCH2, KC), jnp.int32),
        pltpu.VMEM((KC, DH), _f32),
        pltpu.VMEM((K, DH), _f32),
        pltpu.SemaphoreType.DMA,
    ],
)
def _sc_degree(dst_hbm, zf_hbm, ones_hbm,
               deg0_hbm, deg1_hbm,
               deg_s, dst_v, ones_v, stage_v, scsem):
    cid = lax.axis_index("c")
    sid = lax.axis_index("s")

    pltpu.sync_copy(dst_hbm.at[cid * NS + sid], dst_v)
    pltpu.sync_copy(zf_hbm, stage_v)
    pltpu.sync_copy(ones_hbm, ones_v)
    for j in range(RCS):
        c = sid + j * NS

        @pl.when(c < NRC)
        def _():
            pltpu.sync_copy(stage_v, deg_s.at[pl.ds(c * K, K)])

    plsc.subcore_barrier()

    hsc = [pltpu.async_copy(ones_v, deg_s.at[dst_v.at[i]], scsem, add=True)
           for i in range(NCH2)]
    for h in hsc:
        h.wait()

    plsc.subcore_barrier()

    def _wb(out_hbm):
        for j in range(RCS):
            c = sid + j * NS

            @pl.when(c < NRC)
            def _():
                pltpu.sync_copy(deg_s.at[pl.ds(c * K, K)], stage_v)
                pltpu.sync_copy(stage_v, out_hbm.at[pl.ds(c * K, K)])

    @pl.when(cid == 0)
    def _():
        _wb(deg0_hbm)

    @pl.when(cid == 1)
    def _():
        _wb(deg1_hbm)


BLK = 1000


def _tc_body(x_ref, a0_ref, a1_ref, d0_ref, d1_ref, wst_ref, wnt0_ref,
             wnt1_ref, b_ref, o_ref):
    deg = d0_ref[:, 0:1] + d1_ref[:, 0:1]
    inv = 1.0 / jnp.maximum(deg, 1.0)
    m0 = a0_ref[...] * inv
    m1 = a1_ref[...] * inv
    acc = jnp.dot(x_ref[...], wst_ref[...], preferred_element_type=_f32)
    acc = acc + jnp.dot(m0, wnt0_ref[...], preferred_element_type=_f32)
    acc = acc + jnp.dot(m1, wnt1_ref[...], preferred_element_type=_f32)
    o_ref[...] = jnp.maximum(acc + b_ref[...], 0.0)


def _tc_dense(x, a0, a1, d0, d1, wst, wnt0, wnt1, b):
    return pl.pallas_call(
        _tc_body,
        grid=(N // BLK,),
        in_specs=[
            pl.BlockSpec((BLK, D), lambda i: (i, 0)),
            pl.BlockSpec((BLK, DH), lambda i: (i, 0)),
            pl.BlockSpec((BLK, DH), lambda i: (i, 0)),
            pl.BlockSpec((BLK, DH), lambda i: (i, 0)),
            pl.BlockSpec((BLK, DH), lambda i: (i, 0)),
            pl.BlockSpec((D, D), lambda i: (0, 0)),
            pl.BlockSpec((DH, D), lambda i: (0, 0)),
            pl.BlockSpec((DH, D), lambda i: (0, 0)),
            pl.BlockSpec((1, D), lambda i: (0, 0)),
        ],
        out_specs=pl.BlockSpec((BLK, D), lambda i: (i, 0)),
        out_shape=jax.ShapeDtypeStruct((N, D), _f32),
    )(x, a0, a1, d0, d1, wst, wnt0, wnt1, b)


def kernel(node_feats, edge_index, W_self, b_self, W_neigh, b_neigh):
    src = edge_index[0]
    dst = edge_index[1]
    t0 = node_feats[:, :DH]
    t1 = node_feats[:, DH:]
    src_p = jnp.pad(src.reshape(NS, EPS), ((0, 0), (0, EPSP - EPS))
                    ).reshape(NS, NCH, KC)
    dst_p = jnp.pad(dst.reshape(NS, EPS), ((0, 0), (0, EPSP - EPS)),
                    constant_values=N).reshape(NS, NCH, KC)
    dst_d = jnp.pad(dst.reshape(2 * NS, EPS2), ((0, 0), (0, EPS2P - EPS2)),
                    constant_values=N).reshape(2 * NS, NCH2, KC)
    zf = jnp.zeros((KC, DH), _f32)
    zf80 = jnp.zeros((K, DH), _f32)
    ones = jnp.ones((KC, DH), _f32)
    a0, a1 = _sc_feats(t0, t1, src_p, dst_p, zf)
    d0, d1 = _sc_degree(dst_d, zf80, ones)
    wst = W_self.T
    wnt = W_neigh.T
    b = (b_self + b_neigh)[None, :]
    return _tc_dense(node_feats, a0, a1, d0, d1, wst, wnt[:DH], wnt[DH:], b)


# speedup vs baseline: 4.5186x; 1.0428x over previous
import functools

import jax
import jax.numpy as jnp
from jax import lax
from jax.experimental import pallas as pl
from jax.experimental.pallas import tpu as pltpu
from jax.experimental.pallas import tpu_sc as plsc

N = 10000
ND = N + 8
D = 256
DH = D // 2
E = 160000
NS = 16
EPS = E // NS
KC = 128
NCH = 80
EPSP = NCH * KC
NZC = N // KC
NZT = N - NZC * KC
NCHD = NCH // 2

_f32 = jnp.float32


@functools.partial(
    pl.kernel,
    out_type=(
        jax.ShapeDtypeStruct((N, DH), _f32),
        jax.ShapeDtypeStruct((N, DH), _f32),
        jax.ShapeDtypeStruct((N, DH), _f32),
        jax.ShapeDtypeStruct((N, DH), _f32),
    ),
    mesh=plsc.VectorSubcoreMesh(core_axis_name="c", subcore_axis_name="s"),
    scratch_types=[
        pltpu.VMEM_SHARED((ND, DH), _f32),
        pltpu.VMEM((4, KC), jnp.int32),
        pltpu.VMEM((4, KC), jnp.int32),
        pltpu.VMEM((KC, DH), _f32),
        pltpu.VMEM((KC, DH), _f32),
        pltpu.VMEM((KC, DH), _f32),
        pltpu.SemaphoreType.DMA,
        pltpu.SemaphoreType.DMA,
        pltpu.SemaphoreType.DMA,
        pltpu.SemaphoreType.DMA,
        pltpu.SemaphoreType.DMA,
        pltpu.SemaphoreType.DMA,
        pltpu.SemaphoreType.DMA,
        pltpu.SemaphoreType.DMA,
        pltpu.SemaphoreType.DMA,
        pltpu.SemaphoreType.DMA,
        pltpu.SemaphoreType.DMA,
        pltpu.SemaphoreType.DMA,
        pltpu.SemaphoreType.DMA,
    ],
)
def _sc_agg(t0_hbm, t1_hbm, src_hbm, dst_hbm, zf_hbm, ones_hbm,
            out0_hbm, out1_hbm, deg0_hbm, deg1_hbm,
            acc_s, src_c, dst_c, rows_a, rows_b, rows_c,
            sga, sgb, sgc, ss0, ss1, ss2, ss3, sd0, sd1, sd2, sd3, sca, scb):
    cid = lax.axis_index("c")
    sid = lax.axis_index("s")

    gsem = (sga, sgb, sgc)
    ssem = (ss0, ss1, ss2, ss3)
    dsem = (sd0, sd1, sd2, sd3)
    csem = (sca, scb)
    bufs = (rows_a, rows_b, rows_c)

    def _zero():
        for j in range(NZC // NS + 1):
            c = sid + j * NS

            @pl.when(c < NZC)
            def _():
                pltpu.sync_copy(rows_a, acc_s.at[pl.ds(c * KC, KC)])

        @pl.when(sid == 0)
        def _():
            pltpu.sync_copy(rows_a.at[pl.ds(0, NZT)],
                            acc_s.at[pl.ds(NZC * KC, NZT)])

    pltpu.sync_copy(zf_hbm, rows_a)
    _zero()
    plsc.subcore_barrier()

    def _run(table_hbm):
        hs = [None] * NCH
        hd = [None] * NCH
        hg = [None] * NCH
        hc = [None] * NCH
        pltpu.sync_copy(src_hbm.at[sid, 0], src_c.at[0])
        pltpu.sync_copy(dst_hbm.at[sid, 0], dst_c.at[0])
        for j in (1, 2):
            hs[j] = pltpu.async_copy(src_hbm.at[sid, j], src_c.at[j], ssem[j])
            hd[j] = pltpu.async_copy(dst_hbm.at[sid, j], dst_c.at[j], dsem[j])
        hg[0] = pltpu.async_copy(table_hbm.at[src_c.at[0]], bufs[0], gsem[0])
        hs[1].wait()
        hd[1].wait()
        hg[1] = pltpu.async_copy(table_hbm.at[src_c.at[1]], bufs[1], gsem[1])
        for i in range(NCH):
            if i >= 1:
                hc[i - 1].wait()
            if i + 2 < NCH:
                hs[i + 2].wait()
                hd[i + 2].wait()
                hg[i + 2] = pltpu.async_copy(
                    table_hbm.at[src_c.at[(i + 2) % 4]], bufs[(i + 2) % 3],
                    gsem[(i + 2) % 3])
            if i + 3 < NCH:
                hs[i + 3] = pltpu.async_copy(
                    src_hbm.at[sid, i + 3], src_c.at[(i + 3) % 4],
                    ssem[(i + 3) % 4])
                hd[i + 3] = pltpu.async_copy(
                    dst_hbm.at[sid, i + 3], dst_c.at[(i + 3) % 4],
                    dsem[(i + 3) % 4])
            hg[i].wait()
            hc[i] = pltpu.async_copy(bufs[i % 3], acc_s.at[dst_c.at[i % 4]],
                                     csem[i % 2], add=True)
        hc[NCH - 1].wait()

    @pl.when(cid == 0)
    def _():
        _run(t0_hbm)

    @pl.when(cid == 1)
    def _():
        _run(t1_hbm)

    plsc.subcore_barrier()

    def _wb(out_hbm):
        for j in range(NZC // NS + 1):
            c = sid + j * NS

            @pl.when(c < NZC)
            def _():
                pltpu.sync_copy(acc_s.at[pl.ds(c * KC, KC)], rows_a)
                pltpu.sync_copy(rows_a, out_hbm.at[pl.ds(c * KC, KC)])

        @pl.when(sid == 0)
        def _():
            pltpu.sync_copy(acc_s.at[pl.ds(NZC * KC, NZT)],
                            rows_a.at[pl.ds(0, NZT)])
            pltpu.sync_copy(rows_a.at[pl.ds(0, NZT)],
                            out_hbm.at[pl.ds(NZC * KC, NZT)])

    @pl.when(cid == 0)
    def _():
        _wb(out0_hbm)

    @pl.when(cid == 1)
    def _():
        _wb(out1_hbm)

    plsc.subcore_barrier()

    pltpu.sync_copy(zf_hbm, rows_a)
    pltpu.sync_copy(ones_hbm, rows_b)
    _zero()
    plsc.subcore_barrier()

    cbase = cid * NCHD
    hd2 = [None] * NCHD
    hg2 = [None] * NCHD
    pltpu.sync_copy(dst_hbm.at[sid, cbase], dst_c.at[0])
    hd2[1] = pltpu.async_copy(dst_hbm.at[sid, cbase + 1], dst_c.at[1],
                              dsem[1])
    for i in range(NCHD):
        if i >= 1:
            hg2[i - 1].wait()
        if i + 1 < NCHD:
            hd2[i + 1].wait()
        if i + 2 < NCHD:
            hd2[i + 2] = pltpu.async_copy(
                dst_hbm.at[sid, cbase + i + 2], dst_c.at[(i + 2) % 4],
                dsem[(i + 2) % 4])
        hg2[i] = pltpu.async_copy(rows_b, acc_s.at[dst_c.at[i % 4]],
                                  csem[i % 2], add=True)
    hg2[NCHD - 1].wait()

    plsc.subcore_barrier()

    @pl.when(cid == 0)
    def _():
        _wb(deg0_hbm)

    @pl.when(cid == 1)
    def _():
        _wb(deg1_hbm)


BLK = 1000


def _tc_body(x_ref, a0_ref, a1_ref, d0_ref, d1_ref, wst_ref, wnt0_ref,
             wnt1_ref, b_ref, o_ref):
    deg = d0_ref[:, 0:1] + d1_ref[:, 0:1]
    inv = 1.0 / jnp.maximum(deg, 1.0)
    m0 = a0_ref[...] * inv
    m1 = a1_ref[...] * inv
    acc = jnp.dot(x_ref[...], wst_ref[...], preferred_element_type=_f32)
    acc = acc + jnp.dot(m0, wnt0_ref[...], preferred_element_type=_f32)
    acc = acc + jnp.dot(m1, wnt1_ref[...], preferred_element_type=_f32)
    o_ref[...] = jnp.maximum(acc + b_ref[...], 0.0)


def _tc_dense(x, a0, a1, d0, d1, wst, wnt0, wnt1, b):
    return pl.pallas_call(
        _tc_body,
        grid=(N // BLK,),
        in_specs=[
            pl.BlockSpec((BLK, D), lambda i: (i, 0)),
            pl.BlockSpec((BLK, DH), lambda i: (i, 0)),
            pl.BlockSpec((BLK, DH), lambda i: (i, 0)),
            pl.BlockSpec((BLK, DH), lambda i: (i, 0)),
            pl.BlockSpec((BLK, DH), lambda i: (i, 0)),
            pl.BlockSpec((D, D), lambda i: (0, 0)),
            pl.BlockSpec((DH, D), lambda i: (0, 0)),
            pl.BlockSpec((DH, D), lambda i: (0, 0)),
            pl.BlockSpec((1, D), lambda i: (0, 0)),
        ],
        out_specs=pl.BlockSpec((BLK, D), lambda i: (i, 0)),
        out_shape=jax.ShapeDtypeStruct((N, D), _f32),
    )(x, a0, a1, d0, d1, wst, wnt0, wnt1, b)


def kernel(node_feats, edge_index, W_self, b_self, W_neigh, b_neigh):
    src = edge_index[0]
    dst = edge_index[1]
    t0 = node_feats[:, :DH]
    t1 = node_feats[:, DH:]
    src_p = jnp.pad(src.reshape(NS, EPS), ((0, 0), (0, EPSP - EPS))
                    ).reshape(NS, NCH, KC)
    dst_p = jnp.pad(dst.reshape(NS, EPS), ((0, 0), (0, EPSP - EPS)),
                    constant_values=N).reshape(NS, NCH, KC)
    zf = jnp.zeros((KC, DH), _f32)
    ones = jnp.ones((KC, DH), _f32)
    a0, a1, d0, d1 = _sc_agg(t0, t1, src_p, dst_p, zf, ones)
    wst = W_self.T
    wnt = W_neigh.T
    b = (b_self + b_neigh)[None, :]
    return _tc_dense(node_feats, a0, a1, d0, d1, wst, wnt[:DH], wnt[DH:], b)
